# Initial kernel scaffold; baseline (speedup 1.0000x reference)
import jax
import jax.numpy as jnp
from jax.experimental import pallas as pl


def kernel(x, edge_index, pos_edge_index, neg_edge_index, W_g1, b_g1, W_g2, b_g2, W_g3, b_g3, Ws1_self, Ws1_neigh, bs1, Ws2_self, Ws2_neigh, bs2, Ws3_self, Ws3_neigh, bs3, Wp1, bp1, Wp2, bp2):
    raise NotImplementedError("write your pallas kernel here")


# speedup vs baseline: 3.1829x; 3.1829x over previous
import functools

import jax
import jax.numpy as jnp
from jax import lax
from jax.experimental import pallas as pl
from jax.experimental.pallas import tpu as pltpu
from jax.experimental.pallas import tpu_sc as plsc

NNODE = 10000
ACC_ROWS = 10112
TRASH_ROW = 10000
ZROW = 632
F32 = jnp.float32
I32 = jnp.int32


def _mesh():
    return plsc.VectorSubcoreMesh(core_axis_name="c", subcore_axis_name="s")


@functools.lru_cache(None)
def _sc_segsum(n_tab, dc, split_edges, nblk):
    n_out = 2 if split_edges else n_tab
    nbw = nblk // 32 if split_edges else nblk // 16

    def body(*refs):
        tbls = refs[:n_tab]
        srcb, dstb, zeros = refs[n_tab:n_tab + 3]
        outs = refs[n_tab + 3:n_tab + 3 + n_out]
        idx_s, idx_d, gbuf, acc, sem = refs[n_tab + 3 + n_out:]
        c = lax.axis_index("c")
        s = lax.axis_index("s")
        blk0 = (c * 16 + s) * nbw if split_edges else s * nbw
        pltpu.sync_copy(srcb.at[pl.ds(blk0, nbw)], idx_s)
        pltpu.sync_copy(dstb.at[pl.ds(blk0, nbw)], idx_d)

        def run_one(tbl, out):
            pltpu.sync_copy(zeros.at[pl.ds(s * ZROW, ZROW)],
                            acc.at[pl.ds(s * ZROW, ZROW)])
            plsc.subcore_barrier()

            @pl.loop(0, nbw)
            def _(j):
                pltpu.async_copy(tbl.at[idx_s.at[j]], gbuf, sem).wait()
                pltpu.sync_copy(gbuf, acc.at[idx_d.at[j]], add=True)

            plsc.subcore_barrier()
            pltpu.sync_copy(acc.at[pl.ds(s * ZROW, ZROW)],
                            out.at[pl.ds(s * ZROW, ZROW)])
            plsc.subcore_barrier()

        if split_edges:
            @pl.when(c == 0)
            def _():
                run_one(tbls[0], outs[0])

            @pl.when(c == 1)
            def _():
                run_one(tbls[0], outs[1])
        else:
            @pl.when(c == 0)
            def _():
                for i in range(0, n_tab, 2):
                    run_one(tbls[i], outs[i])

            @pl.when(c == 1)
            def _():
                for i in range(1, n_tab, 2):
                    run_one(tbls[i], outs[i])

    return pl.kernel(
        body,
        out_type=[jax.ShapeDtypeStruct((ACC_ROWS, dc), F32)] * n_out,
        mesh=_mesh(),
        scratch_types=[
            pltpu.VMEM((nbw, 128), I32),
            pltpu.VMEM((nbw, 128), I32),
            pltpu.VMEM((128, dc), F32),
            pltpu.VMEM_SHARED((ACC_ROWS, dc), F32),
            pltpu.SemaphoreType.DMA,
        ],
        name=f"sc_segsum_{n_tab}x{dc}_{'split' if split_edges else 'chunk'}",
    )


@functools.lru_cache(None)
def _sc_counts(nblk):
    nbw = nblk // 32

    def body(dstb, zeros, ones_h, out0, out1, idx_d, ones, acc):
        c = lax.axis_index("c")
        s = lax.axis_index("s")
        blk0 = (c * 16 + s) * nbw
        pltpu.sync_copy(dstb.at[pl.ds(blk0, nbw)], idx_d)
        pltpu.sync_copy(ones_h, ones)
        pltpu.sync_copy(zeros.at[pl.ds(s * ZROW, ZROW)],
                        acc.at[pl.ds(s * ZROW, ZROW)])
        plsc.subcore_barrier()

        @pl.loop(0, nbw)
        def _(j):
            pltpu.sync_copy(ones, acc.at[idx_d.at[j]], add=True)

        plsc.subcore_barrier()

        @pl.when(c == 0)
        def _():
            pltpu.sync_copy(acc.at[pl.ds(s * ZROW, ZROW)],
                            out0.at[pl.ds(s * ZROW, ZROW)])

        @pl.when(c == 1)
        def _():
            pltpu.sync_copy(acc.at[pl.ds(s * ZROW, ZROW)],
                            out1.at[pl.ds(s * ZROW, ZROW)])

    return pl.kernel(
        body,
        out_type=[jax.ShapeDtypeStruct((ACC_ROWS, 128), F32)] * 2,
        mesh=_mesh(),
        scratch_types=[
            pltpu.VMEM((nbw, 128), I32),
            pltpu.VMEM((128, 128), F32),
            pltpu.VMEM_SHARED((ACC_ROWS, 128), F32),
        ],
        name="sc_counts",
    )


@functools.lru_cache(None)
def _sc_gather(nblk, dc):
    nbw = nblk // 32

    def body(tbl, idxb, out, idxv, gbuf, sem):
        c = lax.axis_index("c")
        s = lax.axis_index("s")
        w = c * 16 + s
        blk0 = w * nbw
        pltpu.sync_copy(idxb.at[w], idxv)

        @pl.loop(0, nbw)
        def _(j):
            pltpu.async_copy(tbl.at[idxv.at[j]], gbuf, sem).wait()
            pltpu.sync_copy(gbuf, out.at[pl.ds((blk0 + j) * 128, 128)])

    return pl.kernel(
        body,
        out_type=jax.ShapeDtypeStruct((nblk * 128, dc), F32),
        mesh=_mesh(),
        scratch_types=[
            pltpu.VMEM((nbw, 128), I32),
            pltpu.VMEM((128, dc), F32),
            pltpu.SemaphoreType.DMA,
        ],
        name="sc_gather",
    )


MB = 1000
_GRID = NNODE // MB
_DOT = dict(preferred_element_type=F32, precision=lax.Precision.HIGHEST)


def _row_spec(d):
    return pl.BlockSpec((MB, d), lambda i: (i, 0))


def _full_spec(shape):
    nd = len(shape)
    return pl.BlockSpec(shape, lambda i: (0,) * nd)


def _rnorm(c0, c1):
    deg = c0[:, 0:1] + c1[:, 0:1]
    return lax.rsqrt(jnp.maximum(deg, 1.0)), jnp.maximum(deg, 1.0)


def _tc_prep(x, c0, c1):
    def body(x_ref, c0_ref, c1_ref, o0, o1, o2, o3):
        r, _ = _rnorm(c0_ref[...], c1_ref[...])
        xs = x_ref[...] * r
        for k, o in enumerate((o0, o1, o2, o3)):
            o[...] = xs[:, 128 * k:128 * (k + 1)]

    return pl.pallas_call(
        body,
        grid=(_GRID,),
        in_specs=[_row_spec(512), _row_spec(128), _row_spec(128)],
        out_specs=[_row_spec(128)] * 4,
        out_shape=[jax.ShapeDtypeStruct((NNODE, 128), F32)] * 4,
    )(x, c0, c1)


def _tc_gcn1(a, c0, c1, W1, b1, W2):
    def body(a0, a1, a2, a3, c0r, c1r, W1r, b1r, W2r, o0, o1):
        r, _ = _rnorm(c0r[...], c1r[...])
        acat = jnp.concatenate([a0[...], a1[...], a2[...], a3[...]], axis=1)
        h1 = jnp.maximum(jnp.dot(acat * r, W1r[...], **_DOT) + b1r[...], 0.0)
        t2 = jnp.dot(h1, W2r[...], **_DOT) * r
        o0[...] = t2[:, :128]
        o1[...] = t2[:, 128:]

    return pl.pallas_call(
        body,
        grid=(_GRID,),
        in_specs=[_row_spec(128)] * 4 + [_row_spec(128)] * 2 +
                 [_full_spec((512, 1024)), _full_spec((1, 1024)),
                  _full_spec((1024, 256))],
        out_specs=[_row_spec(128)] * 2,
        out_shape=[jax.ShapeDtypeStruct((NNODE, 128), F32)] * 2,
    )(*a, c0, c1, W1, b1, W2)


def _tc_gcn2(a, c0, c1, b2, W3):
    def body(a0, a1, c0r, c1r, b2r, W3r, o0):
        r, _ = _rnorm(c0r[...], c1r[...])
        acat = jnp.concatenate([a0[...], a1[...]], axis=1)
        h2 = jnp.maximum(acat * r + b2r[...], 0.0)
        o0[...] = jnp.dot(h2, W3r[...], **_DOT) * r

    return pl.pallas_call(
        body,
        grid=(_GRID,),
        in_specs=[_row_spec(128)] * 2 + [_row_spec(128)] * 2 +
                 [_full_spec((1, 256)), _full_spec((256, 128))],
        out_specs=_row_spec(128),
        out_shape=jax.ShapeDtypeStruct((NNODE, 128), F32),
    )(*a, c0, c1, b2, W3)


def _tc_gcn3(p0, p1, c0, c1, b3, Wn):
    def body(p0r, p1r, c0r, c1r, b3r, Wnr, oh, ou):
        r, _ = _rnorm(c0r[...], c1r[...])
        h3 = jnp.maximum((p0r[...] + p1r[...]) * r + b3r[...], 0.0)
        oh[...] = h3
        u = jnp.dot(h3, Wnr[...], **_DOT)
        ou[...] = jnp.concatenate([u, jnp.zeros_like(u, shape=(u.shape[0], 64))], axis=1)

    return pl.pallas_call(
        body,
        grid=(_GRID,),
        in_specs=[_row_spec(128)] * 2 + [_row_spec(128)] * 2 +
                 [_full_spec((1, 128)), _full_spec((128, 64))],
        out_specs=[_row_spec(128), _row_spec(128)],
        out_shape=[jax.ShapeDtypeStruct((NNODE, 128), F32),
                   jax.ShapeDtypeStruct((NNODE, 128), F32)],
    )(p0, p1, c0, c1, b3, Wn)


def _tc_sage(h, p0, p1, c0, c1, Wself, b, Wnext, din, dmid, dnext, relu):
    has_next = Wnext is not None

    def body(hr, p0r, p1r, c0r, c1r, Wsr, br, *rest):
        _, cnt = _rnorm(c0r[...], c1r[...])
        agg = (p0r[...] + p1r[...])[:, :dmid] / cnt
        hn = jnp.dot(hr[...], Wsr[...], **_DOT) + agg + br[...]
        if relu:
            hn = jnp.maximum(hn, 0.0)
        if has_next:
            Wnr, oh, ou = rest
            oh[...] = hn
            u = jnp.dot(hn, Wnr[...], **_DOT)
            ou[...] = jnp.concatenate(
                [u, jnp.zeros_like(u, shape=(u.shape[0], 128 - dnext))], axis=1)
        else:
            (oh,) = rest
            oh[...] = jnp.concatenate(
                [hn, jnp.zeros_like(hn, shape=(hn.shape[0], 128 - dmid))], axis=1)

    in_specs = [_row_spec(din), _row_spec(128), _row_spec(128),
                _row_spec(128), _row_spec(128),
                _full_spec((din, dmid)), _full_spec((1, dmid))]
    args = [h, p0, p1, c0, c1, Wself, b]
    if has_next:
        in_specs.append(_full_spec((dmid, dnext)))
        args.append(Wnext)
        return pl.pallas_call(
            body,
            grid=(_GRID,),
            in_specs=in_specs,
            out_specs=[_row_spec(dmid), _row_spec(128)],
            out_shape=[jax.ShapeDtypeStruct((NNODE, dmid), F32),
                       jax.ShapeDtypeStruct((NNODE, 128), F32)],
        )(*args)
    return pl.pallas_call(
        body,
        grid=(_GRID,),
        in_specs=in_specs,
        out_specs=_row_spec(128),
        out_shape=jax.ShapeDtypeStruct((NNODE, 128), F32),
    )(*args)


def _tc_pred(g, Wp1, bp1, Wp2, bp2, p):
    pb = 2000
    npb = p // pb

    def body(gu, gv, nu, nv, W1r, b1r, W2r, b2r, opos, oneg):
        def mlp(hu, hv):
            f = jnp.concatenate([hu, hv, hu * hv, jnp.abs(hu - hv)], axis=1)
            z = jnp.maximum(jnp.dot(f, W1r[...], **_DOT) + b1r[...], 0.0)
            return jnp.dot(z, W2r[...], **_DOT) + b2r[...]

        opos[...] = mlp(gu[...][:, :32], gv[...][:, :32])
        oneg[...] = mlp(nu[...][:, :32], nv[...][:, :32])

    eview = lambda off: pl.BlockSpec((pb, 128), lambda i, o=off: (i + o, 0))
    return pl.pallas_call(
        body,
        grid=(npb,),
        in_specs=[eview(0), eview(npb), eview(2 * npb), eview(3 * npb),
                  _full_spec((128, 128)), _full_spec((1, 128)),
                  _full_spec((128, 1)), _full_spec((1, 1))],
        out_specs=[pl.BlockSpec((pb, 1), lambda i: (i, 0))] * 2,
        out_shape=[jax.ShapeDtypeStruct((p, 1), F32)] * 2,
    )(g, g, g, g, Wp1, bp1, Wp2, bp2)


def kernel(x, edge_index, pos_edge_index, neg_edge_index,
           W_g1, b_g1, W_g2, b_g2, W_g3, b_g3,
           Ws1_self, Ws1_neigh, bs1, Ws2_self, Ws2_neigh, bs2,
           Ws3_self, Ws3_neigh, bs3, Wp1, bp1, Wp2, bp2):
    e = edge_index.shape[1]
    epad = -(-e // 4096) * 4096
    nblk = epad // 128
    src = jnp.concatenate(
        [edge_index[0], jnp.zeros((epad - e,), I32)]).reshape(nblk, 128)
    dst = jnp.concatenate(
        [edge_index[1], jnp.full((epad - e,), TRASH_ROW, I32)]).reshape(nblk, 128)

    z128 = jnp.zeros((ACC_ROWS, 128), F32)

    c0, c1 = _sc_counts(nblk)(dst, z128, jnp.ones((128, 128), F32))

    xp = _tc_prep(x, c0, c1)
    a1 = _sc_segsum(4, 128, False, nblk)(*xp, src, dst, z128)
    t2 = _tc_gcn1(a1, c0, c1, W_g1, b_g1.reshape(1, -1), W_g2)
    a2 = _sc_segsum(2, 128, False, nblk)(*t2, src, dst, z128)
    t3 = _tc_gcn2(a2, c0, c1, b_g2.reshape(1, -1), W_g3)
    p3 = _sc_segsum(1, 128, True, nblk)(t3, src, dst, z128)
    h3, u4 = _tc_gcn3(p3[0], p3[1], c0, c1, b_g3.reshape(1, -1), Ws1_neigh)

    q4 = _sc_segsum(1, 128, True, nblk)(u4, src, dst, z128)
    h4, u5 = _tc_sage(h3, q4[0], q4[1], c0, c1, Ws1_self,
                      bs1.reshape(1, -1), Ws2_neigh, 128, 64, 32, True)
    q5 = _sc_segsum(1, 128, True, nblk)(u5, src, dst, z128)
    h5, u6 = _tc_sage(h4, q5[0], q5[1], c0, c1, Ws2_self,
                      bs2.reshape(1, -1), Ws3_neigh, 64, 32, 32, True)
    q6 = _sc_segsum(1, 128, True, nblk)(u6, src, dst, z128)
    h6 = _tc_sage(h5, q6[0], q6[1], c0, c1, Ws3_self,
                  bs3.reshape(1, -1), None, 32, 32, 0, False)

    p = pos_edge_index.shape[1]
    gpad = -(-4 * p // 4096) * 4096
    gnblk = gpad // 128
    idx_all = jnp.concatenate(
        [pos_edge_index[0], pos_edge_index[1],
         neg_edge_index[0], neg_edge_index[1],
         jnp.zeros((gpad - 4 * p,), I32)]).reshape(32, gnblk // 32, 128)
    g = _sc_gather(gnblk, 128)(h6, idx_all)

    pos, neg = _tc_pred(g, Wp1, bp1.reshape(1, -1), Wp2, bp2.reshape(1, -1), p)
    return (pos, neg)


# speedup vs baseline: 7.4314x; 2.3348x over previous
import functools

import jax
import jax.numpy as jnp
from jax import lax
from jax.experimental import pallas as pl
from jax.experimental.pallas import tpu as pltpu
from jax.experimental.pallas import tpu_sc as plsc

NNODE = 10000
ACC_ROWS = 10112
TRASH_ROW = 10000
ZROW = 632
F32 = jnp.float32
I32 = jnp.int32
NBUF = 4


def _mesh():
    return plsc.VectorSubcoreMesh(core_axis_name="c", subcore_axis_name="s")


@functools.lru_cache(None)
def _sc_segsum(n_tab, dc, split_edges, nblk):
    n_out = 2 if split_edges else n_tab
    nbw = nblk // 32 if split_edges else nblk // 16
    iseg = 40
    nseg = nbw // iseg
    assert dc % 128 == 0
    nbuf = 2

    def body(*refs):
        tbls = refs[:n_tab]
        srcb, dstb, zeros = refs[n_tab:n_tab + 3]
        outs = refs[n_tab + 3:n_tab + 3 + n_out]
        rest = refs[n_tab + 3 + n_out:]
        idx_s, idx_d = rest[0], rest[1]
        gb = rest[2:2 + nbuf]
        acc = rest[2 + nbuf]
        sems = rest[3 + nbuf:]
        c = lax.axis_index("c")
        s = lax.axis_index("s")
        blk0 = (c * 16 + s) * nbw if split_edges else s * nbw

        def run_one(tbl, out):
            pltpu.sync_copy(zeros.at[pl.ds(s * ZROW, ZROW)],
                            acc.at[pl.ds(s * ZROW, ZROW)])
            plsc.subcore_barrier()

            for seg in range(nseg):
                pltpu.sync_copy(srcb.at[pl.ds(blk0 + seg * iseg, iseg)], idx_s)
                pltpu.sync_copy(dstb.at[pl.ds(blk0 + seg * iseg, iseg)], idx_d)

                @pl.loop(0, iseg // nbuf)
                def _(i):
                    j = i * nbuf
                    cps = [pltpu.async_copy(tbl.at[idx_s.at[j + k]],
                                            gb[k], sems[k])
                           for k in range(nbuf)]
                    for k in range(nbuf):
                        cps[k].wait()
                        pltpu.sync_copy(gb[k], acc.at[idx_d.at[j + k]],
                                        add=True)

            plsc.subcore_barrier()
            pltpu.sync_copy(acc.at[pl.ds(s * ZROW, ZROW)],
                            out.at[pl.ds(s * ZROW, ZROW)])
            plsc.subcore_barrier()

        if split_edges:
            @pl.when(c == 0)
            def _():
                run_one(tbls[0], outs[0])

            @pl.when(c == 1)
            def _():
                run_one(tbls[0], outs[1])
        else:
            @pl.when(c == 0)
            def _():
                for i in range(0, n_tab, 2):
                    run_one(tbls[i], outs[i])

            @pl.when(c == 1)
            def _():
                for i in range(1, n_tab, 2):
                    run_one(tbls[i], outs[i])

    return pl.kernel(
        body,
        out_type=[jax.ShapeDtypeStruct((ACC_ROWS, dc), F32)] * n_out,
        mesh=_mesh(),
        scratch_types=[
            pltpu.VMEM((iseg, 128), I32),
            pltpu.VMEM((iseg, 128), I32),
        ] + [pltpu.VMEM((128, dc), F32)] * nbuf + [
            pltpu.VMEM_SHARED((ACC_ROWS, dc), F32),
        ] + [pltpu.SemaphoreType.DMA] * nbuf,
        name=f"sc_segsum_{n_tab}x{dc}_{'split' if split_edges else 'chunk'}",
    )


@functools.lru_cache(None)
def _sc_counts(nblk):
    nbw = nblk // 32

    def body(dstb, zeros, ones_h, out0, out1, idx_d, ones, acc):
        c = lax.axis_index("c")
        s = lax.axis_index("s")
        blk0 = (c * 16 + s) * nbw
        pltpu.sync_copy(dstb.at[pl.ds(blk0, nbw)], idx_d)
        pltpu.sync_copy(ones_h, ones)
        pltpu.sync_copy(zeros.at[pl.ds(s * ZROW, ZROW)],
                        acc.at[pl.ds(s * ZROW, ZROW)])
        plsc.subcore_barrier()

        @pl.loop(0, nbw)
        def _(j):
            pltpu.sync_copy(ones, acc.at[idx_d.at[j]], add=True)

        plsc.subcore_barrier()

        @pl.when(c == 0)
        def _():
            pltpu.sync_copy(acc.at[pl.ds(s * ZROW, ZROW)],
                            out0.at[pl.ds(s * ZROW, ZROW)])

        @pl.when(c == 1)
        def _():
            pltpu.sync_copy(acc.at[pl.ds(s * ZROW, ZROW)],
                            out1.at[pl.ds(s * ZROW, ZROW)])

    return pl.kernel(
        body,
        out_type=[jax.ShapeDtypeStruct((ACC_ROWS, 128), F32)] * 2,
        mesh=_mesh(),
        scratch_types=[
            pltpu.VMEM((nbw, 128), I32),
            pltpu.VMEM((128, 128), F32),
            pltpu.VMEM_SHARED((ACC_ROWS, 128), F32),
        ],
        name="sc_counts",
    )


@functools.lru_cache(None)
def _sc_gather(nblk, dc):
    nbw = nblk // 32

    def body(tbl, idxb, out, idxv, gb0, tsh):
        c = lax.axis_index("c")
        s = lax.axis_index("s")
        w = c * 16 + s
        blk0 = w * nbw
        pltpu.sync_copy(idxb.at[w], idxv)
        nfull = NNODE // ZROW

        @pl.when(s < nfull)
        def _():
            pltpu.sync_copy(tbl.at[pl.ds(s * ZROW, ZROW)],
                            tsh.at[pl.ds(s * ZROW, ZROW)])

        @pl.when(s == nfull)
        def _():
            pltpu.sync_copy(
                tbl.at[pl.ds(nfull * ZROW, NNODE - nfull * ZROW)],
                tsh.at[pl.ds(nfull * ZROW, NNODE - nfull * ZROW)])

        plsc.subcore_barrier()

        @pl.loop(0, nbw)
        def _(j):
            pltpu.sync_copy(tsh.at[idxv.at[j]], gb0)
            pltpu.sync_copy(gb0, out.at[pl.ds((blk0 + j) * 128, 128)])

    return pl.kernel(
        body,
        out_type=jax.ShapeDtypeStruct((nblk * 128, dc), F32),
        mesh=_mesh(),
        scratch_types=[
            pltpu.VMEM((nbw, 128), I32),
            pltpu.VMEM((128, dc), F32),
            pltpu.VMEM_SHARED((ACC_ROWS, dc), F32),
        ],
        name="sc_gather",
    )


MB = 1000
_GRID = NNODE // MB
_DOT = dict(preferred_element_type=F32, precision=lax.Precision.HIGHEST)


def _row_spec(d):
    return pl.BlockSpec((MB, d), lambda i: (i, 0))


def _full_spec(shape):
    nd = len(shape)
    return pl.BlockSpec(shape, lambda i: (0,) * nd)


def _tc_prep(x, c0, c1):
    def body(x_ref, c0_ref, c1_ref, o0, o1, o2, o3, o_r, o_c):
        deg = jnp.maximum(c0_ref[:, :1] + c1_ref[:, :1], 1.0)
        r = lax.rsqrt(deg)
        xs = x_ref[...] * r
        for k, o in enumerate((o0, o1, o2, o3)):
            o[...] = xs[:, 128 * k:128 * (k + 1)]
        o_r[...] = jnp.broadcast_to(r, (MB, 16))
        o_c[...] = jnp.broadcast_to(deg, (MB, 16))

    return pl.pallas_call(
        body,
        grid=(_GRID,),
        in_specs=[_row_spec(512), _row_spec(128), _row_spec(128)],
        out_specs=[_row_spec(128)] * 4 + [_row_spec(16)] * 2,
        out_shape=[jax.ShapeDtypeStruct((NNODE, 128), F32)] * 4 +
                  [jax.ShapeDtypeStruct((NNODE, 16), F32)] * 2,
    )(x, c0, c1)


def _tc_gcn1(a, rr, W1, b1, W2):
    def body(a0, a1, a2, a3, rref, W1r, b1r, W2r, o0, o1):
        r = rref[:, :1]
        acat = jnp.concatenate([a0[...], a1[...], a2[...], a3[...]], axis=1)
        h1 = jnp.maximum(jnp.dot(acat * r, W1r[...], **_DOT) + b1r[...], 0.0)
        t2 = jnp.dot(h1, W2r[...], **_DOT) * r
        o0[...] = t2[:, :128]
        o1[...] = t2[:, 128:]

    return pl.pallas_call(
        body,
        grid=(_GRID,),
        in_specs=[_row_spec(128)] * 4 + [_row_spec(16)] +
                 [_full_spec((512, 1024)), _full_spec((1, 1024)),
                  _full_spec((1024, 256))],
        out_specs=[_row_spec(128)] * 2,
        out_shape=[jax.ShapeDtypeStruct((NNODE, 128), F32)] * 2,
    )(*a, rr, W1, b1, W2)


def _tc_gcn2(a, rr, b2, W3):
    def body(a0, a1, rref, b2r, W3r, o0):
        r = rref[:, :1]
        acat = jnp.concatenate([a0[...], a1[...]], axis=1)
        h2 = jnp.maximum(acat * r + b2r[...], 0.0)
        o0[...] = jnp.dot(h2, W3r[...], **_DOT) * r

    return pl.pallas_call(
        body,
        grid=(_GRID,),
        in_specs=[_row_spec(128)] * 2 + [_row_spec(16)] +
                 [_full_spec((1, 256)), _full_spec((256, 128))],
        out_specs=_row_spec(128),
        out_shape=jax.ShapeDtypeStruct((NNODE, 128), F32),
    )(*a, rr, b2, W3)


def _tc_gcn3(p0, p1, rr, b3, Wn):
    def body(p0r, p1r, rref, b3r, Wnr, oh, ou):
        r = rref[:, :1]
        h3 = jnp.maximum((p0r[...] + p1r[...]) * r + b3r[...], 0.0)
        oh[...] = h3
        ou[...] = jnp.dot(h3, Wnr[...], **_DOT)

    return pl.pallas_call(
        body,
        grid=(_GRID,),
        in_specs=[_row_spec(128)] * 2 + [_row_spec(16)] +
                 [_full_spec((1, 128)), _full_spec((128, 128))],
        out_specs=[_row_spec(128), _row_spec(128)],
        out_shape=[jax.ShapeDtypeStruct((NNODE, 128), F32),
                   jax.ShapeDtypeStruct((NNODE, 128), F32)],
    )(p0, p1, rr, b3, Wn)


def _tc_sage(h, p0, p1, cc, Wself, b, Wnext, din, dmid, relu):
    has_next = Wnext is not None

    def body(hr, p0r, p1r, ccr, Wsr, br, *rest):
        cnt = ccr[:, :1]
        agg = (p0r[...] + p1r[...])[:, :dmid] / cnt
        hn = jnp.dot(hr[...], Wsr[...], **_DOT) + agg + br[...]
        if relu:
            hn = jnp.maximum(hn, 0.0)
        if has_next:
            Wnr, oh, ou = rest
            oh[...] = hn
            ou[...] = jnp.dot(hn, Wnr[...], **_DOT)
        else:
            (oh,) = rest
            oh[...] = jnp.concatenate(
                [hn, jnp.zeros((MB, 128 - dmid), F32)], axis=1)

    in_specs = [_row_spec(din), _row_spec(128), _row_spec(128),
                _row_spec(16),
                _full_spec((din, dmid)), _full_spec((1, dmid))]
    args = [h, p0, p1, cc, Wself, b]
    if has_next:
        in_specs.append(_full_spec((dmid, 128)))
        args.append(Wnext)
        return pl.pallas_call(
            body,
            grid=(_GRID,),
            in_specs=in_specs,
            out_specs=[_row_spec(dmid), _row_spec(128)],
            out_shape=[jax.ShapeDtypeStruct((NNODE, dmid), F32),
                       jax.ShapeDtypeStruct((NNODE, 128), F32)],
        )(*args)
    return pl.pallas_call(
        body,
        grid=(_GRID,),
        in_specs=in_specs,
        out_specs=_row_spec(128),
        out_shape=jax.ShapeDtypeStruct((NNODE, 128), F32),
    )(*args)


def _tc_pred(g, Wp1, bp1, Wp2, bp2, p):
    pb = 2000
    npb = p // pb

    def body(gu, gv, nu, nv, W1r, b1r, W2r, b2r, opos, oneg):
        def mlp(hu, hv):
            f = jnp.concatenate([hu, hv, hu * hv, jnp.abs(hu - hv)], axis=1)
            z = jnp.maximum(jnp.dot(f, W1r[...], **_DOT) + b1r[...], 0.0)
            return jnp.dot(z, W2r[...], **_DOT) + b2r[...]

        opos[...] = mlp(gu[...][:, :32], gv[...][:, :32])
        oneg[...] = mlp(nu[...][:, :32], nv[...][:, :32])

    eview = lambda off: pl.BlockSpec((pb, 128), lambda i, o=off: (i + o, 0))
    return pl.pallas_call(
        body,
        grid=(npb,),
        in_specs=[eview(0), eview(npb), eview(2 * npb), eview(3 * npb),
                  _full_spec((128, 128)), _full_spec((1, 128)),
                  _full_spec((128, 1)), _full_spec((1, 1))],
        out_specs=[pl.BlockSpec((pb, 1), lambda i: (i, 0))] * 2,
        out_shape=[jax.ShapeDtypeStruct((p, 1), F32)] * 2,
    )(g, g, g, g, Wp1, bp1, Wp2, bp2)


def kernel(x, edge_index, pos_edge_index, neg_edge_index,
           W_g1, b_g1, W_g2, b_g2, W_g3, b_g3,
           Ws1_self, Ws1_neigh, bs1, Ws2_self, Ws2_neigh, bs2,
           Ws3_self, Ws3_neigh, bs3, Wp1, bp1, Wp2, bp2):
    e = edge_index.shape[1]
    epad = -(-e // 16384) * 16384
    nblk = epad // 128
    npadi = jnp.arange(epad - e, dtype=I32)
    src = jnp.concatenate(
        [edge_index[0], npadi % NNODE]).reshape(nblk, 128)
    dst = jnp.concatenate(
        [edge_index[1],
         TRASH_ROW + npadi % (ACC_ROWS - NNODE)]).reshape(nblk, 128)

    z128 = jnp.zeros((ACC_ROWS, 128), F32)
    Wn1 = jnp.pad(Ws1_neigh, ((0, 0), (0, 128 - Ws1_neigh.shape[1])))
    Wn2 = jnp.pad(Ws2_neigh, ((0, 0), (0, 128 - Ws2_neigh.shape[1])))
    Wn3 = jnp.pad(Ws3_neigh, ((0, 0), (0, 128 - Ws3_neigh.shape[1])))

    ones_tbl = jnp.ones((NNODE, 128), F32)
    c0, c1 = _sc_segsum(1, 128, True, nblk)(ones_tbl, src, dst, z128)

    *xp, rr, cc = _tc_prep(x, c0, c1)
    a1 = list(_sc_segsum(2, 128, False, nblk)(xp[0], xp[1], src, dst, z128)) \
       + list(_sc_segsum(2, 128, False, nblk)(xp[2], xp[3], src, dst, z128))
    t2 = _tc_gcn1(a1, rr, W_g1, b_g1.reshape(1, -1), W_g2)
    a2 = _sc_segsum(2, 128, False, nblk)(*t2, src, dst, z128)
    t3 = _tc_gcn2(a2, rr, b_g2.reshape(1, -1), W_g3)
    p3 = _sc_segsum(1, 128, True, nblk)(t3, src, dst, z128)
    h3, u4 = _tc_gcn3(p3[0], p3[1], rr, b_g3.reshape(1, -1), Wn1)

    q4 = _sc_segsum(1, 128, True, nblk)(u4, src, dst, z128)
    h4, u5 = _tc_sage(h3, q4[0], q4[1], cc, Ws1_self,
                      bs1.reshape(1, -1), Wn2, 128, 64, True)
    q5 = _sc_segsum(1, 128, True, nblk)(u5, src, dst, z128)
    h5, u6 = _tc_sage(h4, q5[0], q5[1], cc, Ws2_self,
                      bs2.reshape(1, -1), Wn3, 64, 32, True)
    q6 = _sc_segsum(1, 128, True, nblk)(u6, src, dst, z128)
    h6 = _tc_sage(h5, q6[0], q6[1], cc, Ws3_self,
                  bs3.reshape(1, -1), None, 32, 32, False)

    p = pos_edge_index.shape[1]
    gpad = -(-4 * p // 16384) * 16384
    gnblk = gpad // 128
    gpadi = jnp.arange(gpad - 4 * p, dtype=I32)
    idx_all = jnp.concatenate(
        [pos_edge_index[0], pos_edge_index[1],
         neg_edge_index[0], neg_edge_index[1],
         gpadi % NNODE]).reshape(32, gnblk // 32, 128)
    g = _sc_gather(gnblk, 128)(h6, idx_all)

    pos, neg = _tc_pred(g, Wp1, bp1.reshape(1, -1), Wp2, bp2.reshape(1, -1), p)
    return (pos, neg)
